# baseline (speedup 1.0000x reference)
import functools

import jax
import jax.numpy as jnp
from jax import lax
from jax.experimental import pallas as pl
from jax.experimental.pallas import tpu as pltpu
from jax.experimental.pallas import tpu_sc as plsc

_B, _TOPK, _D, _V, _C = 4096, 16, 64, 64, 100000
_BT = 1024
_L = 4096
_NCB = (_C + _L - 1) // _L
_NEG = float("-inf")


_CH = _L // 128
_BIG = 2 ** 30


def _fold(v, g):
    w = 128
    while w > _TOPK:
        h = w // 2
        av, bv = v[:, :h], v[:, h:]
        ag, bg = g[:, :h], g[:, h:]
        take = (bv > av) | ((bv == av) & (bg < ag))
        v = jnp.where(take, bv, av)
        g = jnp.where(take, bg, ag)
        w = h
    t = g
    while t.shape[1] < 128:
        t = jnp.concatenate([t, t], axis=1)
    return v, g, t


def _merge16_t(rvt, rit, cv, ci):
    vals = jnp.concatenate([rvt, cv.T], axis=0)
    gidx = jnp.concatenate([rit, ci.T], axis=0)
    out_v, out_i = [], []
    for _ in range(_TOPK):
        m = jnp.max(vals, axis=0, keepdims=True)
        hit = vals == m
        s = jnp.min(jnp.where(hit, gidx, _BIG), axis=0, keepdims=True)
        out_v.append(m)
        out_i.append(s)
        vals = jnp.where(hit & (gidx == s), _NEG, vals)
    return jnp.concatenate(out_v, axis=0), jnp.concatenate(out_i, axis=0)


def _topk_body(q_ref, k_ref, scores_ref, idx_ref, x_ref, rv_ref, ri_ref,
               m_ref, s_ref, t_ref):
    j = pl.program_id(1)

    @pl.when(j == 0)
    def _init():
        rv_ref[...] = jnp.full((_TOPK, _BT), _NEG, jnp.float32)
        ri_ref[...] = jnp.zeros((_TOPK, _BT), jnp.int32)
        t_ref[...] = jnp.full((_BT, 1), _NEG, jnp.float32)

    sim = lax.dot_general(q_ref[...], k_ref[...], (((1,), (1,)), ((), ())),
                          preferred_element_type=jnp.float32)

    @pl.when(j < _NCB - 1)
    def _store_full():
        x_ref[...] = sim

    @pl.when(j == _NCB - 1)
    def _store_masked():
        col = j * _L + lax.broadcasted_iota(jnp.int32, (_BT, _L), 1)
        x_ref[...] = jnp.where(col < _C, sim, _NEG)

    lane = lax.broadcasted_iota(jnp.int32, (_BT, 128), 1)

    m0 = x_ref[:, 0:128]
    s0 = jnp.zeros((_BT, 128), jnp.int32)
    for c in range(1, _CH):
        xc = x_ref[:, c * 128:(c + 1) * 128]
        upd = xc > m0
        m0 = jnp.where(upd, xc, m0)
        s0 = jnp.where(upd, c, s0)
    m_ref[...] = m0
    s_ref[...] = s0

    def cond(go):
        return go

    def body(_):
        m = m_ref[...]
        sel = s_ref[...]
        g = j * _L + sel * 128 + lane
        cv, ci, wt = _fold(m, g)
        m2 = jnp.full((_BT, 128), _NEG, jnp.float32)
        s2 = jnp.zeros((_BT, 128), jnp.int32)
        for c in range(_CH):
            xc = x_ref[:, c * 128:(c + 1) * 128]
            xc = jnp.where((sel == c) & (g == wt), _NEG, xc)
            x_ref[:, c * 128:(c + 1) * 128] = xc
            upd = xc > m2
            m2 = jnp.where(upd, xc, m2)
            s2 = jnp.where(upd, c, s2)
        rvt, rit = _merge16_t(rv_ref[...], ri_ref[...], cv, ci)
        rv_ref[...] = rvt
        ri_ref[...] = rit
        t_ref[...] = rvt[_TOPK - 1:_TOPK, :].T
        m_ref[...] = m2
        s_ref[...] = s2
        return jnp.any(m2 >= t_ref[...])

    lax.while_loop(cond, body, jnp.any(m0 >= t_ref[...]))

    @pl.when(j == _NCB - 1)
    def _fin():
        scores_ref[...] = rv_ref[...].T
        idx_ref[...] = ri_ref[...].T


def _topk_call(qn, kn):
    return pl.pallas_call(
        _topk_body,
        grid=(_B // _BT, _NCB),
        in_specs=[
            pl.BlockSpec((_BT, _D), lambda i, j: (i, 0)),
            pl.BlockSpec((_L, _D), lambda i, j: (j, 0)),
        ],
        out_specs=[
            pl.BlockSpec((_BT, _TOPK), lambda i, j: (i, 0)),
            pl.BlockSpec((_BT, _TOPK), lambda i, j: (i, 0)),
        ],
        out_shape=[jax.ShapeDtypeStruct((_B, _TOPK), jnp.float32),
                   jax.ShapeDtypeStruct((_B, _TOPK), jnp.int32)],
        scratch_shapes=[pltpu.VMEM((_BT, _L), jnp.float32),
                        pltpu.VMEM((_TOPK, _BT), jnp.float32),
                        pltpu.VMEM((_TOPK, _BT), jnp.int32),
                        pltpu.VMEM((_BT, 128), jnp.float32),
                        pltpu.VMEM((_BT, 128), jnp.int32),
                        pltpu.VMEM((_BT, 1), jnp.float32)],
        compiler_params=pltpu.CompilerParams(
            dimension_semantics=("parallel", "arbitrary")),
    )(qn, kn)


def _gather_values(values, idx_flat):
    info = plsc.get_sparse_core_info()
    nw = info.num_cores * info.num_subcores
    n = idx_flat.shape[0]
    per_w = n // nw
    ch = 128
    n_ch = per_w // ch
    mesh = plsc.VectorSubcoreMesh(core_axis_name="c", subcore_axis_name="s")

    @functools.partial(
        pl.kernel, mesh=mesh,
        out_type=jax.ShapeDtypeStruct((n, _V), jnp.float32),
        scratch_types=[pltpu.VMEM((ch,), jnp.int32),
                       pltpu.VMEM((ch, _V), jnp.float32),
                       pltpu.SemaphoreType.DMA],
        compiler_params=pltpu.CompilerParams(use_tc_tiling_on_sc=False),
    )
    def k(table_hbm, idx_hbm, out_hbm, idx_v, rows_v, sem):
        wid = lax.axis_index("s") * info.num_cores + lax.axis_index("c")
        base = wid * per_w
        for t in range(n_ch):
            off = base + t * ch
            pltpu.sync_copy(idx_hbm.at[pl.ds(off, ch)], idx_v)
            pltpu.async_copy(table_hbm.at[idx_v], rows_v, sem).wait()
            pltpu.sync_copy(rows_v, out_hbm.at[pl.ds(off, ch)])

    return k(values, idx_flat)


def kernel(query, keys, values, W, b, top_k):
    del top_k
    q = query @ W.T + b
    qn = q / jnp.maximum(jnp.linalg.norm(q, axis=-1, keepdims=True), 1e-12)
    kn = keys / jnp.maximum(jnp.linalg.norm(keys, axis=-1, keepdims=True), 1e-12)
    scores, indices = _topk_call(qn.astype(jnp.bfloat16), kn.astype(jnp.bfloat16))
    retrieved = _gather_values(values, indices.reshape(-1))
    return retrieved.reshape(_B, _TOPK, _V), scores


# speedup vs baseline: 1.2657x; 1.2657x over previous
import functools

import jax
import jax.numpy as jnp
from jax import lax
from jax.experimental import pallas as pl
from jax.experimental.pallas import tpu as pltpu
from jax.experimental.pallas import tpu_sc as plsc

_B, _TOPK, _D, _V, _C = 4096, 16, 64, 64, 100000
_BT = 1024
_L = 4096
_NCB = (_C + _L - 1) // _L
_NEG = float("-inf")


_CH = _L // 128
_BIG = 2 ** 30


def _fold(v, g):
    w = 128
    while w > _TOPK:
        h = w // 2
        av, bv = v[:, :h], v[:, h:]
        ag, bg = g[:, :h], g[:, h:]
        take = (bv > av) | ((bv == av) & (bg < ag))
        v = jnp.where(take, bv, av)
        g = jnp.where(take, bg, ag)
        w = h
    t = g
    while t.shape[1] < 128:
        t = jnp.concatenate([t, t], axis=1)
    return v, g, t


def _merge16_t(rvt, rit, cv, ci):
    vals = jnp.concatenate([rvt, cv.T], axis=0)
    gidx = jnp.concatenate([rit, ci.T], axis=0)
    out_v, out_i = [], []
    for _ in range(_TOPK):
        m = jnp.max(vals, axis=0, keepdims=True)
        hit = vals == m
        s = jnp.min(jnp.where(hit, gidx, _BIG), axis=0, keepdims=True)
        out_v.append(m)
        out_i.append(s)
        vals = jnp.where(hit & (gidx == s), _NEG, vals)
    return jnp.concatenate(out_v, axis=0), jnp.concatenate(out_i, axis=0)


def _topk_body(q_ref, k_ref, scores_ref, idx_ref, x_ref, rv_ref, ri_ref,
               t_ref):
    j = pl.program_id(1)

    @pl.when(j == 0)
    def _init():
        rv_ref[...] = jnp.full((_TOPK, _BT), _NEG, jnp.float32)
        ri_ref[...] = jnp.zeros((_TOPK, _BT), jnp.int32)
        t_ref[...] = jnp.full((_BT, 1), _NEG, jnp.float32)

    sim = lax.dot_general(q_ref[...], k_ref[...], (((1,), (1,)), ((), ())),
                          preferred_element_type=jnp.float32)
    col = j * _L + lax.broadcasted_iota(jnp.int32, (_BT, _L), 1)
    sim = jnp.where(col < _C, sim, _NEG)
    x_ref[...] = sim

    lane = lax.broadcasted_iota(jnp.int32, (_BT, 128), 1)

    m0 = sim[:, 0:128]
    s0 = jnp.zeros((_BT, 128), jnp.int32)
    for c in range(1, _CH):
        xc = sim[:, c * 128:(c + 1) * 128]
        upd = xc > m0
        m0 = jnp.where(upd, xc, m0)
        s0 = jnp.where(upd, c, s0)

    def cond(carry):
        return carry[2]

    def body(carry):
        m, sel, _ = carry
        g = j * _L + sel * 128 + lane
        cv, ci, wt = _fold(m, g)
        m2 = jnp.full((_BT, 128), _NEG, jnp.float32)
        s2 = jnp.zeros((_BT, 128), jnp.int32)
        for c in range(_CH):
            xc = x_ref[:, c * 128:(c + 1) * 128]
            xc = jnp.where((sel == c) & (g == wt), _NEG, xc)
            x_ref[:, c * 128:(c + 1) * 128] = xc
            upd = xc > m2
            m2 = jnp.where(upd, xc, m2)
            s2 = jnp.where(upd, c, s2)
        rvt, rit = _merge16_t(rv_ref[...], ri_ref[...], cv, ci)
        rv_ref[...] = rvt
        ri_ref[...] = rit
        t15 = rvt[_TOPK - 1:_TOPK, :].T
        t_ref[...] = t15
        return m2, s2, jnp.any(m2 >= t15)

    lax.while_loop(cond, body, (m0, s0, jnp.any(m0 >= t_ref[...])))

    @pl.when(j == _NCB - 1)
    def _fin():
        scores_ref[...] = rv_ref[...].T
        idx_ref[...] = ri_ref[...].T


def _topk_call(qn, kn):
    return pl.pallas_call(
        _topk_body,
        grid=(_B // _BT, _NCB),
        in_specs=[
            pl.BlockSpec((_BT, _D), lambda i, j: (i, 0)),
            pl.BlockSpec((_L, _D), lambda i, j: (j, 0)),
        ],
        out_specs=[
            pl.BlockSpec((_BT, _TOPK), lambda i, j: (i, 0)),
            pl.BlockSpec((_BT, _TOPK), lambda i, j: (i, 0)),
        ],
        out_shape=[jax.ShapeDtypeStruct((_B, _TOPK), jnp.float32),
                   jax.ShapeDtypeStruct((_B, _TOPK), jnp.int32)],
        scratch_shapes=[pltpu.VMEM((_BT, _L), jnp.float32),
                        pltpu.VMEM((_TOPK, _BT), jnp.float32),
                        pltpu.VMEM((_TOPK, _BT), jnp.int32),
                        pltpu.VMEM((_BT, 1), jnp.float32)],
        compiler_params=pltpu.CompilerParams(
            dimension_semantics=("parallel", "arbitrary")),
    )(qn, kn)


def _gather_values(values, idx_flat):
    info = plsc.get_sparse_core_info()
    nw = info.num_cores * info.num_subcores
    n = idx_flat.shape[0]
    per_w = n // nw
    ch = 128
    n_ch = per_w // ch
    mesh = plsc.VectorSubcoreMesh(core_axis_name="c", subcore_axis_name="s")

    @functools.partial(
        pl.kernel, mesh=mesh,
        out_type=jax.ShapeDtypeStruct((n, _V), jnp.float32),
        scratch_types=[pltpu.VMEM((ch,), jnp.int32),
                       pltpu.VMEM((ch, _V), jnp.float32),
                       pltpu.SemaphoreType.DMA],
        compiler_params=pltpu.CompilerParams(use_tc_tiling_on_sc=False),
    )
    def k(table_hbm, idx_hbm, out_hbm, idx_v, rows_v, sem):
        wid = lax.axis_index("s") * info.num_cores + lax.axis_index("c")
        base = wid * per_w
        for t in range(n_ch):
            off = base + t * ch
            pltpu.sync_copy(idx_hbm.at[pl.ds(off, ch)], idx_v)
            pltpu.async_copy(table_hbm.at[idx_v], rows_v, sem).wait()
            pltpu.sync_copy(rows_v, out_hbm.at[pl.ds(off, ch)])

    return k(values, idx_flat)


def kernel(query, keys, values, W, b, top_k):
    del top_k
    q = query @ W.T + b
    qn = q / jnp.maximum(jnp.linalg.norm(q, axis=-1, keepdims=True), 1e-12)
    kn = keys / jnp.maximum(jnp.linalg.norm(keys, axis=-1, keepdims=True), 1e-12)
    scores, indices = _topk_call(qn.astype(jnp.bfloat16), kn.astype(jnp.bfloat16))
    retrieved = _gather_values(values, indices.reshape(-1))
    return retrieved.reshape(_B, _TOPK, _V), scores
